# baseline (speedup 1.0000x reference)
import functools

import jax
import jax.numpy as jnp
from jax import lax
from jax.experimental import pallas as pl
from jax.experimental.pallas import tpu as pltpu
from jax.experimental.pallas import tpu_sc as plsc

N_NODES = 10000
N_EDGES = 160000
D = 256
H = D // 2
NS = 16
EPT = N_EDGES // NS
B = 104
KR = 97
KB = KR + 2
NPAD = 10240
RPT = NPAD // NS


def _sc_segment_sum(x2, idx_all, zer):
    mesh = plsc.VectorSubcoreMesh(core_axis_name="c", subcore_axis_name="s")

    @functools.partial(
        pl.kernel,
        out_type=jax.ShapeDtypeStruct((2, NPAD, H), jnp.float32),
        mesh=mesh,
        scratch_types=[
            pltpu.VMEM((2, B), jnp.int32),
            pltpu.VMEM((2, B), jnp.int32),
            pltpu.VMEM((2, B), jnp.int32),
            pltpu.VMEM((B, H), jnp.float32),
            pltpu.VMEM((B, H), jnp.float32),
            pltpu.VMEM((B, H), jnp.float32),
            pltpu.VMEM_SHARED((NPAD, H), jnp.float32),
            pltpu.SemaphoreType.DMA,
            pltpu.SemaphoreType.DMA,
            pltpu.SemaphoreType.DMA,
            pltpu.SemaphoreType.DMA,
            pltpu.SemaphoreType.DMA,
            pltpu.SemaphoreType.DMA,
            pltpu.SemaphoreType.DMA,
            pltpu.SemaphoreType.DMA,
            pltpu.SemaphoreType.DMA,
        ],
    )
    def sc_accum(x2_hbm, idx_hbm, zer_hbm, out_hbm,
                 ib0, ib1, ib2, buf0, buf1, buf2, s_sh,
                 si0, si1, si2, sg0, sg1, sg2, ss0, ss1, ss2):
        c = lax.axis_index("c")
        t = lax.axis_index("s")
        ib = (ib0, ib1, ib2)
        buf = (buf0, buf1, buf2)
        si = (si0, si1, si2)
        sg = (sg0, sg1, sg2)
        ss = (ss0, ss1, ss2)
        pltpu.sync_copy(zer_hbm, s_sh.at[pl.ds(t * RPT, RPT)])
        plsc.subcore_barrier()

        def ring_step(k, a, b, cc, with_prev_scatter_wait=True):
            pltpu.make_async_copy(idx_hbm.at[c, t, k + 1], ib[b], si[b]).wait()
            pltpu.async_copy(x2_hbm.at[ib[b].at[0]], buf[b], sg[b])
            pltpu.make_async_copy(x2_hbm.at[ib[a].at[0]], buf[a], sg[a]).wait()
            pltpu.async_copy(buf[a], s_sh.at[ib[a].at[1]], ss[a], add=True)
            if with_prev_scatter_wait:
                pltpu.make_async_copy(buf[cc], s_sh.at[ib[cc].at[1]],
                                      ss[cc]).wait()
            pltpu.async_copy(idx_hbm.at[c, t, k + 2], ib[cc], si[cc])

        pltpu.sync_copy(idx_hbm.at[c, t, 0], ib0)
        pltpu.async_copy(x2_hbm.at[ib0.at[0]], buf0, sg0)
        pltpu.async_copy(idx_hbm.at[c, t, 1], ib1, si1)
        ring_step(0, 0, 1, 2, with_prev_scatter_wait=False)

        def step(j, carry):
            k0 = 3 * j + 1
            ring_step(k0, 1, 2, 0)
            ring_step(k0 + 1, 2, 0, 1)
            ring_step(k0 + 2, 0, 1, 2)
            return carry

        lax.fori_loop(0, (KR - 1) // 3, step, 0)
        kl = (KR - 1) % 3
        kg = KR % 3
        ki = (KR + 1) % 3
        pltpu.make_async_copy(x2_hbm.at[ib[kg].at[0]], buf[kg], sg[kg]).wait()
        pltpu.make_async_copy(buf[kl], s_sh.at[ib[kl].at[1]], ss[kl]).wait()
        pltpu.make_async_copy(idx_hbm.at[c, t, KR + 1], ib[ki], si[ki]).wait()
        plsc.subcore_barrier()
        pltpu.sync_copy(s_sh.at[pl.ds(t * RPT, RPT)],
                        out_hbm.at[c, pl.ds(t * RPT, RPT)])

    return sc_accum(x2, idx_all, zer)


def _tc_finish(x, s0, s1, wt, bias2):
    blk = 2000
    grid = (N_NODES // blk,)

    def body(x_ref, s0_ref, s1_ref, wt_ref, b_ref, o_ref):
        xs = x_ref[...] * jnp.concatenate([s0_ref[...], s1_ref[...]], axis=-1)
        o_ref[...] = (jnp.dot(xs, wt_ref[...],
                              preferred_element_type=jnp.float32)
                      + b_ref[...])

    return pl.pallas_call(
        body,
        grid=grid,
        in_specs=[
            pl.BlockSpec((blk, D), lambda i: (i, 0)),
            pl.BlockSpec((blk, H), lambda i: (i, 0)),
            pl.BlockSpec((blk, H), lambda i: (i, 0)),
            pl.BlockSpec((D, D), lambda i: (0, 0)),
            pl.BlockSpec((1, D), lambda i: (0, 0)),
        ],
        out_specs=pl.BlockSpec((blk, D), lambda i: (i, 0)),
        out_shape=jax.ShapeDtypeStruct((N_NODES, D), jnp.float32),
    )(x, s0, s1, wt, bias2)


def kernel(x, edge_index, weight, bias):
    row = edge_index[0].astype(jnp.int32)
    col = edge_index[1].astype(jnp.int32)
    x2 = x.reshape(2 * N_NODES, H)
    npad = KB * B - EPT
    colp = jnp.concatenate(
        [col.reshape(NS, EPT),
         jnp.zeros((NS, npad), jnp.int32)], axis=1)
    rowp = jnp.concatenate(
        [row.reshape(NS, EPT),
         jnp.full((NS, npad), NPAD - 1, jnp.int32)], axis=1)
    gidx = jnp.stack([colp * 2, colp * 2 + 1])
    sidx = jnp.broadcast_to(rowp, (2, NS, KB * B))
    idx_all = jnp.stack(
        [gidx.reshape(2, NS, KB, B), sidx.reshape(2, NS, KB, B)],
        axis=3)
    zer = jnp.zeros((RPT, H), dtype=jnp.float32)

    s = _sc_segment_sum(x2, idx_all, zer)

    wt = weight.T
    bias2 = bias[None, :]
    return _tc_finish(x, s[0], s[1], wt, bias2)
